# baseline (speedup 1.0000x reference)
import functools

import jax
import jax.numpy as jnp
from jax import lax
from jax.experimental import pallas as pl
from jax.experimental.pallas import tpu as pltpu
from jax.experimental.pallas import tpu_sc as plsc

B = 16
P1 = 6890
P2 = 4096
L = 16
P1PAD = 6912
BIG = 3.0e38
PAD_COORD = 1.0e6

NSC = 4
NTC = B - NSC
W = 32 // NSC
BPS = NSC // 2
ROWS = P1PAD // W
NBLK = ROWS // L
U = 9 if NBLK % 9 == 0 else (4 if NBLK % 4 == 0 else 3)
NG = NBLK // U
JW = P2 // W

_mesh = plsc.VectorSubcoreMesh(core_axis_name="c", subcore_axis_name="s")


def _rne_bf16(x):
    u = plsc.bitcast(x, jnp.uint32)
    r = (u + jnp.uint32(0x7FFF) + ((u >> jnp.uint32(16)) & jnp.uint32(1)))
    r = r & jnp.uint32(0xFFFF0000)
    return plsc.bitcast(r, jnp.float32)


@functools.partial(
    pl.kernel,
    mesh=_mesh,
    compiler_params=pltpu.CompilerParams(needs_layout_passes=False),
    out_type=jax.ShapeDtypeStruct((32, 4, L), jnp.float32),
    scratch_types=[
        pltpu.VMEM((4, ROWS), jnp.float32),
        pltpu.VMEM((P2,), jnp.float32),
        pltpu.VMEM((P2,), jnp.float32),
        pltpu.VMEM((P2,), jnp.float32),
        pltpu.VMEM((P2,), jnp.float32),
        pltpu.VMEM((ROWS,), jnp.float32),
        pltpu.VMEM((P2,), jnp.float32),
        pltpu.VMEM((P2 * L,), jnp.float32),
        pltpu.VMEM((P2,), jnp.float32),
        pltpu.VMEM((JW,), jnp.float32),
        pltpu.VMEM_SHARED((L, P2), jnp.float32),
        pltpu.VMEM((4, L), jnp.float32),
    ],
)
def _sc_loss(smpl_hbm, obj_hbm, out_hbm,
             s_buf, oa, ob, oc, ocmv, sn, on, cml, cpart, pbuf, shared, outv):
    c = lax.axis_index("c")
    s = lax.axis_index("s")
    w = c * 16 + s
    batch = c * BPS + s // W
    sl_id = s % W

    pltpu.sync_copy(smpl_hbm.at[w], s_buf)
    pltpu.sync_copy(obj_hbm.at[batch, 0], oa)
    pltpu.sync_copy(obj_hbm.at[batch, 1], ob)
    pltpu.sync_copy(obj_hbm.at[batch, 2], oc)
    pltpu.sync_copy(obj_hbm.at[batch, 3], ocmv)

    def sn_body(i, _):
        sl = pl.ds(i * L, L)
        x = s_buf[0, sl]
        y = s_buf[1, sl]
        z = s_buf[2, sl]
        sn[sl] = x * x + y * y + z * z
        s_buf[0, sl] = _rne_bf16(x)
        s_buf[1, sl] = _rne_bf16(y)
        s_buf[2, sl] = _rne_bf16(z)
        return 0

    lax.fori_loop(0, NBLK, sn_body, 0)

    def on_body(j, _):
        sl = pl.ds(j * L, L)
        x = oa[sl]
        y = ob[sl]
        z = oc[sl]
        on[sl] = x * x + y * y + z * z
        oa[sl] = -2.0 * _rne_bf16(x)
        ob[sl] = -2.0 * _rne_bf16(y)
        oc[sl] = -2.0 * _rne_bf16(z)
        return 0

    lax.fori_loop(0, P2 // L, on_body, 0)

    big = jnp.full((L,), BIG, jnp.float32)

    def cml_init(j, _):
        cml[pl.ds(j * L, L)] = big
        return 0

    lax.fori_loop(0, P2, cml_init, 0)

    def group_body(g, accs):
        row_acc, den_acc = accs
        base = g * (U * L)
        xs, ys, zs, sns = [], [], [], []
        for u in range(U):
            sl = pl.ds(base + u * L, L)
            xs.append(s_buf[0, sl])
            ys.append(s_buf[1, sl])
            zs.append(s_buf[2, sl])
            sns.append(sn[sl])

        def jc_body(jc, rms):
            sl = pl.ds(jc * L, L)
            avec = oa[sl]
            bvec = ob[sl]
            cvec = oc[sl]
            svec = on[sl]
            for l in range(L):
                a = avec[l]
                b = bvec[l]
                cc = cvec[l]
                sj = svec[l]
                d = [None] * U
                for u in range(U):
                    d[u] = (sns[u] + sj) + xs[u] * a + ys[u] * b + zs[u] * cc
                m = d[0]
                for u in range(1, U):
                    m = jnp.minimum(m, d[u])
                csl = pl.ds(jc * (L * L) + l * L, L)
                cml[csl] = jnp.minimum(cml[csl], m)
                rms = tuple(jnp.minimum(rms[u], d[u]) for u in range(U))
            return rms

        rms = lax.fori_loop(0, P2 // L, jc_body, (big,) * U)
        for u in range(U):
            sl = pl.ds(base + u * L, L)
            scm = s_buf[3, sl]
            row_acc = row_acc + scm * jnp.maximum(rms[u], 0.0)
            den_acc = den_acc + scm
        return row_acc, den_acc

    zero = jnp.zeros((L,), jnp.float32)
    row_acc, rowden_acc = lax.fori_loop(0, NG, group_body, (zero, zero))

    lanes = lax.iota(jnp.int32, L)

    def red_body(jg, _):
        acc = zero
        for l in range(L):
            row = cml[pl.ds((jg * L + l) * L, L)]
            acc = jnp.where(lanes == l, jnp.min(row), acc)
        cpart[pl.ds(jg * L, L)] = acc
        return 0

    lax.fori_loop(0, P2 // L, red_body, 0)

    pltpu.sync_copy(cpart, shared.at[s])
    plsc.subcore_barrier()
    jbase = sl_id * JW
    grp = (s // W) * W

    for p in range(W):
        peer = grp + p

        @pl.when(peer != s)
        def _():
            pltpu.sync_copy(shared.at[peer, pl.ds(jbase, JW)], pbuf)

            def min_body(k, _):
                sl = pl.ds(jbase + k * L, L)
                cpart[sl] = jnp.minimum(cpart[sl], pbuf[pl.ds(k * L, L)])
                return 0

            lax.fori_loop(0, JW // L, min_body, 0)

    def comb_body(k, accs):
        obj_acc, oden_acc = accs
        sl = pl.ds(jbase + k * L, L)
        v = jnp.maximum(cpart[sl], 0.0)
        ocm_vec = ocmv[sl]
        return obj_acc + ocm_vec * v, oden_acc + ocm_vec

    obj_acc, objden_acc = lax.fori_loop(0, JW // L, comb_body, (zero, zero))

    outv[0, :] = row_acc
    outv[1, :] = rowden_acc
    outv[2, :] = obj_acc
    outv[3, :] = objden_acc
    pltpu.sync_copy(outv, out_hbm.at[w])


TI = 768
NI = P1PAD // TI


def _tc_body(x_ref, y_ref, scm_ref, ocm_ref, out_ref, colmin_ref):
    ni = pl.program_id(1)

    x = x_ref[0]
    y = y_ref[0]
    x2 = jnp.sum(x * x, axis=0)[:, None]
    y2 = jnp.sum(y * y, axis=0)[None, :]
    xy = jax.lax.dot_general(
        x, y, (((0,), (0,)), ((), ())), preferred_element_type=jnp.float32
    )
    d = x2 + y2 - 2.0 * xy

    @pl.when(ni == 0)
    def _init():
        colmin_ref[...] = jnp.full_like(colmin_ref, jnp.inf)
        out_ref[0, 0, 0] = 0.0
        out_ref[0, 0, 1] = 0.0
        out_ref[0, 0, 2] = 0.0
        out_ref[0, 0, 3] = 0.0

    scm = scm_ref[0, 0]
    rowmin = jnp.maximum(jnp.min(d, axis=1), 0.0)
    out_ref[0, 0, 0] += jnp.sum(scm * rowmin)
    out_ref[0, 0, 1] += jnp.sum(scm)

    colmin_ref[...] = jnp.minimum(colmin_ref[...], jnp.min(d, axis=0, keepdims=True))

    @pl.when(ni == NI - 1)
    def _fini():
        ocm = ocm_ref[0, 0]
        colmin = jnp.maximum(colmin_ref[0], 0.0)
        out_ref[0, 0, 2] = jnp.sum(ocm * colmin)
        out_ref[0, 0, 3] = jnp.sum(ocm)


def _tc_loss(xpad, object_v, scm, ocm, nb):
    parts = pl.pallas_call(
        _tc_body,
        grid=(nb, NI),
        in_specs=[
            pl.BlockSpec((1, 3, TI), lambda b, i: (b, 0, i)),
            pl.BlockSpec((1, 3, P2), lambda b, i: (b, 0, 0)),
            pl.BlockSpec((1, 1, TI), lambda b, i: (b, 0, i)),
            pl.BlockSpec((1, 1, P2), lambda b, i: (b, 0, 0)),
        ],
        out_specs=pl.BlockSpec((1, 1, 4), lambda b, i: (b, 0, 0),
                               memory_space=pltpu.SMEM),
        out_shape=jax.ShapeDtypeStruct((nb, 1, 4), jnp.float32),
        scratch_shapes=[pltpu.VMEM((1, P2), jnp.float32)],
    )(xpad, object_v, scm, ocm)
    parts = parts[:, 0]
    return jnp.sum(parts[:, 0] / (parts[:, 1] + 1e-6)
                   + parts[:, 2] / (parts[:, 3] + 1e-6))


@jax.jit
def kernel(smpl_v, object_v, smpl_contact_maps, object_contact_maps):
    xt = smpl_v.transpose(0, 2, 1)
    xT = jnp.pad(xt, ((0, 0), (0, 0), (0, P1PAD - P1)),
                 constant_values=PAD_COORD)
    yT = object_v.transpose(0, 2, 1)
    scm = jnp.pad(smpl_contact_maps[..., 0], ((0, 0), (0, P1PAD - P1)))
    ocm = object_contact_maps[..., 0]

    smpl4 = jnp.concatenate([xT[NTC:], scm[NTC:, None, :]], axis=1)
    smpl4 = smpl4.reshape(NSC, 4, W, ROWS).transpose(0, 2, 1, 3)
    smpl4 = smpl4.reshape(2, 16, 4, ROWS).reshape(32, 4, ROWS)

    obj4 = jnp.concatenate([yT[NTC:], ocm[NTC:, None, :]], axis=1)

    parts = _sc_loss(smpl4, obj4)
    parts = parts.sum(axis=2).reshape(NSC, W, 4).sum(axis=1)
    loss_sc = jnp.sum(parts[:, 0] / (parts[:, 1] + 1e-6)
                      + parts[:, 2] / (parts[:, 3] + 1e-6))

    loss_tc = _tc_loss(xT[:NTC], yT[:NTC],
                       scm[:NTC, None, :], ocm[:NTC, None, :], NTC)

    return (loss_tc + loss_sc) / B


# speedup vs baseline: 1.2510x; 1.2510x over previous
import functools

import jax
import jax.numpy as jnp
from jax import lax
from jax.experimental import pallas as pl
from jax.experimental.pallas import tpu as pltpu
from jax.experimental.pallas import tpu_sc as plsc

B = 16
P1 = 6890
P2 = 4096
L = 16
P1PAD = 6912
BIG = 3.0e38
PAD_COORD = 1.0e6

NSC = 4
NTC = B - NSC
W = 32 // NSC
BPS = NSC // 2
ROWS = P1PAD // W
NBLK = ROWS // L
U = 6 if NBLK % 6 == 0 else (4 if NBLK % 4 == 0 else 3)
NG = NBLK // U
JW = P2 // W

_mesh = plsc.VectorSubcoreMesh(core_axis_name="c", subcore_axis_name="s")


def _rne_bf16(x):
    u = plsc.bitcast(x, jnp.uint32)
    r = (u + jnp.uint32(0x7FFF) + ((u >> jnp.uint32(16)) & jnp.uint32(1)))
    r = r & jnp.uint32(0xFFFF0000)
    return plsc.bitcast(r, jnp.float32)


@functools.partial(
    pl.kernel,
    mesh=_mesh,
    compiler_params=pltpu.CompilerParams(needs_layout_passes=False),
    out_type=jax.ShapeDtypeStruct((32, 4, L), jnp.float32),
    scratch_types=[
        pltpu.VMEM((4, ROWS), jnp.float32),
        pltpu.VMEM((P2,), jnp.float32),
        pltpu.VMEM((P2,), jnp.float32),
        pltpu.VMEM((P2,), jnp.float32),
        pltpu.VMEM((P2,), jnp.float32),
        pltpu.VMEM((ROWS,), jnp.float32),
        pltpu.VMEM((P2,), jnp.float32),
        pltpu.VMEM((P2 * L,), jnp.float32),
        pltpu.VMEM((P2,), jnp.float32),
        pltpu.VMEM((JW,), jnp.float32),
        pltpu.VMEM_SHARED((L, P2), jnp.float32),
        pltpu.VMEM((4, L), jnp.float32),
    ],
)
def _sc_loss(smpl_hbm, obj_hbm, out_hbm,
             s_buf, oa, ob, oc, ocmv, sn, on, cml, cpart, pbuf, shared, outv):
    c = lax.axis_index("c")
    s = lax.axis_index("s")
    w = c * 16 + s
    batch = c * BPS + s // W
    sl_id = s % W

    pltpu.sync_copy(smpl_hbm.at[w], s_buf)
    pltpu.sync_copy(obj_hbm.at[batch, 0], oa)
    pltpu.sync_copy(obj_hbm.at[batch, 1], ob)
    pltpu.sync_copy(obj_hbm.at[batch, 2], oc)
    pltpu.sync_copy(obj_hbm.at[batch, 3], ocmv)

    def sn_body(i, _):
        sl = pl.ds(i * L, L)
        x = s_buf[0, sl]
        y = s_buf[1, sl]
        z = s_buf[2, sl]
        sn[sl] = x * x + y * y + z * z
        s_buf[0, sl] = _rne_bf16(x)
        s_buf[1, sl] = _rne_bf16(y)
        s_buf[2, sl] = _rne_bf16(z)
        return 0

    lax.fori_loop(0, NBLK, sn_body, 0)

    def on_body(j, _):
        sl = pl.ds(j * L, L)
        x = oa[sl]
        y = ob[sl]
        z = oc[sl]
        on[sl] = x * x + y * y + z * z
        oa[sl] = -2.0 * _rne_bf16(x)
        ob[sl] = -2.0 * _rne_bf16(y)
        oc[sl] = -2.0 * _rne_bf16(z)
        return 0

    lax.fori_loop(0, P2 // L, on_body, 0)

    big = jnp.full((L,), BIG, jnp.float32)

    def cml_init(j, _):
        cml[pl.ds(j * L, L)] = big
        return 0

    lax.fori_loop(0, P2, cml_init, 0)

    def group_body(g, accs):
        row_acc, den_acc = accs
        base = g * (U * L)
        xs, ys, zs, sns = [], [], [], []
        for u in range(U):
            sl = pl.ds(base + u * L, L)
            xs.append(s_buf[0, sl])
            ys.append(s_buf[1, sl])
            zs.append(s_buf[2, sl])
            sns.append(sn[sl])

        def jc_body(jc, rms):
            sl = pl.ds(jc * L, L)
            avec = oa[sl]
            bvec = ob[sl]
            cvec = oc[sl]
            svec = on[sl]
            for l in range(L):
                a = avec[l]
                b = bvec[l]
                cc = cvec[l]
                sj = svec[l]
                d = [None] * U
                for u in range(U):
                    d[u] = (sns[u] + sj) + xs[u] * a + ys[u] * b + zs[u] * cc
                m = d[0]
                for u in range(1, U):
                    m = jnp.minimum(m, d[u])
                csl = pl.ds(jc * (L * L) + l * L, L)
                cml[csl] = jnp.minimum(cml[csl], m)
                rms = tuple(jnp.minimum(rms[u], d[u]) for u in range(U))
            return rms

        rms = plsc.parallel_loop(0, P2 // L, carry=(big,) * U, unroll=2)(jc_body)
        for u in range(U):
            sl = pl.ds(base + u * L, L)
            scm = s_buf[3, sl]
            row_acc = row_acc + scm * jnp.maximum(rms[u], 0.0)
            den_acc = den_acc + scm
        return row_acc, den_acc

    zero = jnp.zeros((L,), jnp.float32)
    row_acc, rowden_acc = lax.fori_loop(0, NG, group_body, (zero, zero))

    lanes = lax.iota(jnp.int32, L)

    def red_body(jg, _):
        acc = zero
        for l in range(L):
            row = cml[pl.ds((jg * L + l) * L, L)]
            acc = jnp.where(lanes == l, jnp.min(row), acc)
        cpart[pl.ds(jg * L, L)] = acc
        return 0

    lax.fori_loop(0, P2 // L, red_body, 0)

    pltpu.sync_copy(cpart, shared.at[s])
    plsc.subcore_barrier()
    jbase = sl_id * JW
    grp = (s // W) * W

    for p in range(W):
        peer = grp + p

        @pl.when(peer != s)
        def _():
            pltpu.sync_copy(shared.at[peer, pl.ds(jbase, JW)], pbuf)

            def min_body(k, _):
                sl = pl.ds(jbase + k * L, L)
                cpart[sl] = jnp.minimum(cpart[sl], pbuf[pl.ds(k * L, L)])
                return 0

            lax.fori_loop(0, JW // L, min_body, 0)

    def comb_body(k, accs):
        obj_acc, oden_acc = accs
        sl = pl.ds(jbase + k * L, L)
        v = jnp.maximum(cpart[sl], 0.0)
        ocm_vec = ocmv[sl]
        return obj_acc + ocm_vec * v, oden_acc + ocm_vec

    obj_acc, objden_acc = lax.fori_loop(0, JW // L, comb_body, (zero, zero))

    outv[0, :] = row_acc
    outv[1, :] = rowden_acc
    outv[2, :] = obj_acc
    outv[3, :] = objden_acc
    pltpu.sync_copy(outv, out_hbm.at[w])


TI = 768
NI = P1PAD // TI


def _tc_body(x_ref, y_ref, scm_ref, ocm_ref, out_ref, colmin_ref):
    ni = pl.program_id(1)

    x = x_ref[0]
    y = y_ref[0]
    x2 = jnp.sum(x * x, axis=0)[:, None]
    y2 = jnp.sum(y * y, axis=0)[None, :]
    xy = jax.lax.dot_general(
        x, y, (((0,), (0,)), ((), ())), preferred_element_type=jnp.float32
    )
    d = x2 + y2 - 2.0 * xy

    @pl.when(ni == 0)
    def _init():
        colmin_ref[...] = jnp.full_like(colmin_ref, jnp.inf)
        out_ref[0, 0, 0] = 0.0
        out_ref[0, 0, 1] = 0.0
        out_ref[0, 0, 2] = 0.0
        out_ref[0, 0, 3] = 0.0

    scm = scm_ref[0, 0]
    rowmin = jnp.maximum(jnp.min(d, axis=1), 0.0)
    out_ref[0, 0, 0] += jnp.sum(scm * rowmin)
    out_ref[0, 0, 1] += jnp.sum(scm)

    colmin_ref[...] = jnp.minimum(colmin_ref[...], jnp.min(d, axis=0, keepdims=True))

    @pl.when(ni == NI - 1)
    def _fini():
        ocm = ocm_ref[0, 0]
        colmin = jnp.maximum(colmin_ref[0], 0.0)
        out_ref[0, 0, 2] = jnp.sum(ocm * colmin)
        out_ref[0, 0, 3] = jnp.sum(ocm)


def _tc_loss(xpad, object_v, scm, ocm, nb):
    parts = pl.pallas_call(
        _tc_body,
        grid=(nb, NI),
        in_specs=[
            pl.BlockSpec((1, 3, TI), lambda b, i: (b, 0, i)),
            pl.BlockSpec((1, 3, P2), lambda b, i: (b, 0, 0)),
            pl.BlockSpec((1, 1, TI), lambda b, i: (b, 0, i)),
            pl.BlockSpec((1, 1, P2), lambda b, i: (b, 0, 0)),
        ],
        out_specs=pl.BlockSpec((1, 1, 4), lambda b, i: (b, 0, 0),
                               memory_space=pltpu.SMEM),
        out_shape=jax.ShapeDtypeStruct((nb, 1, 4), jnp.float32),
        scratch_shapes=[pltpu.VMEM((1, P2), jnp.float32)],
    )(xpad, object_v, scm, ocm)
    parts = parts[:, 0]
    return jnp.sum(parts[:, 0] / (parts[:, 1] + 1e-6)
                   + parts[:, 2] / (parts[:, 3] + 1e-6))


@jax.jit
def kernel(smpl_v, object_v, smpl_contact_maps, object_contact_maps):
    xt = smpl_v.transpose(0, 2, 1)
    xT = jnp.pad(xt, ((0, 0), (0, 0), (0, P1PAD - P1)),
                 constant_values=PAD_COORD)
    yT = object_v.transpose(0, 2, 1)
    scm = jnp.pad(smpl_contact_maps[..., 0], ((0, 0), (0, P1PAD - P1)))
    ocm = object_contact_maps[..., 0]

    smpl4 = jnp.concatenate([xT[NTC:], scm[NTC:, None, :]], axis=1)
    smpl4 = smpl4.reshape(NSC, 4, W, ROWS).transpose(0, 2, 1, 3)
    smpl4 = smpl4.reshape(2, 16, 4, ROWS).reshape(32, 4, ROWS)

    obj4 = jnp.concatenate([yT[NTC:], ocm[NTC:, None, :]], axis=1)

    parts = _sc_loss(smpl4, obj4)
    parts = parts.sum(axis=2).reshape(NSC, W, 4).sum(axis=1)
    loss_sc = jnp.sum(parts[:, 0] / (parts[:, 1] + 1e-6)
                      + parts[:, 2] / (parts[:, 3] + 1e-6))

    loss_tc = _tc_loss(xT[:NTC], yT[:NTC],
                       scm[:NTC, None, :], ocm[:NTC, None, :], NTC)

    return (loss_tc + loss_sc) / B


# speedup vs baseline: 1.2593x; 1.0066x over previous
import functools

import jax
import jax.numpy as jnp
from jax import lax
from jax.experimental import pallas as pl
from jax.experimental.pallas import tpu as pltpu
from jax.experimental.pallas import tpu_sc as plsc

B = 16
P1 = 6890
P2 = 4096
L = 16
P1PAD = 6912
BIG = 3.0e38
PAD_COORD = 1.0e6

NSC = 4
NTC = B - NSC
W = 32 // NSC
BPS = NSC // 2
ROWS = P1PAD // W
NBLK = ROWS // L
U = 6 if NBLK % 6 == 0 else (4 if NBLK % 4 == 0 else 3)
NG = NBLK // U
JW = P2 // W

_mesh = plsc.VectorSubcoreMesh(core_axis_name="c", subcore_axis_name="s")


def _rne_bf16(x):
    u = plsc.bitcast(x, jnp.uint32)
    r = (u + jnp.uint32(0x7FFF) + ((u >> jnp.uint32(16)) & jnp.uint32(1)))
    r = r & jnp.uint32(0xFFFF0000)
    return plsc.bitcast(r, jnp.float32)


@functools.partial(
    pl.kernel,
    mesh=_mesh,
    compiler_params=pltpu.CompilerParams(needs_layout_passes=False),
    out_type=jax.ShapeDtypeStruct((32, 4, L), jnp.float32),
    scratch_types=[
        pltpu.VMEM((4, ROWS), jnp.float32),
        pltpu.VMEM((P2,), jnp.float32),
        pltpu.VMEM((P2,), jnp.float32),
        pltpu.VMEM((P2,), jnp.float32),
        pltpu.VMEM((P2,), jnp.float32),
        pltpu.VMEM((ROWS,), jnp.float32),
        pltpu.VMEM((P2,), jnp.float32),
        pltpu.VMEM((P2 * L,), jnp.float32),
        pltpu.VMEM((P2,), jnp.float32),
        pltpu.VMEM((JW,), jnp.float32),
        pltpu.VMEM_SHARED((L, P2), jnp.float32),
        pltpu.VMEM((4, L), jnp.float32),
    ],
)
def _sc_loss(smpl_hbm, obj_hbm, out_hbm,
             s_buf, oa, ob, oc, ocmv, sn, on, cml, cpart, pbuf, shared, outv):
    c = lax.axis_index("c")
    s = lax.axis_index("s")
    w = c * 16 + s
    batch = c * BPS + s // W
    sl_id = s % W

    pltpu.sync_copy(smpl_hbm.at[w], s_buf)
    pltpu.sync_copy(obj_hbm.at[batch, 0], oa)
    pltpu.sync_copy(obj_hbm.at[batch, 1], ob)
    pltpu.sync_copy(obj_hbm.at[batch, 2], oc)
    pltpu.sync_copy(obj_hbm.at[batch, 3], ocmv)

    def sn_body(i, _):
        sl = pl.ds(i * L, L)
        x = s_buf[0, sl]
        y = s_buf[1, sl]
        z = s_buf[2, sl]
        sn[sl] = x * x + y * y + z * z
        s_buf[0, sl] = _rne_bf16(x)
        s_buf[1, sl] = _rne_bf16(y)
        s_buf[2, sl] = _rne_bf16(z)
        return 0

    lax.fori_loop(0, NBLK, sn_body, 0)

    def on_body(j, _):
        sl = pl.ds(j * L, L)
        x = oa[sl]
        y = ob[sl]
        z = oc[sl]
        on[sl] = x * x + y * y + z * z
        oa[sl] = -2.0 * _rne_bf16(x)
        ob[sl] = -2.0 * _rne_bf16(y)
        oc[sl] = -2.0 * _rne_bf16(z)
        return 0

    lax.fori_loop(0, P2 // L, on_body, 0)

    big = jnp.full((L,), BIG, jnp.float32)

    def cml_init(j, _):
        cml[pl.ds(j * L, L)] = big
        return 0

    lax.fori_loop(0, P2, cml_init, 0)

    def group_body(g, accs):
        row_acc, den_acc = accs
        base = g * (U * L)
        xs, ys, zs, sns = [], [], [], []
        for u in range(U):
            sl = pl.ds(base + u * L, L)
            xs.append(s_buf[0, sl])
            ys.append(s_buf[1, sl])
            zs.append(s_buf[2, sl])
            sns.append(sn[sl])

        def jc_body(jc, rms):
            sl = pl.ds(jc * L, L)
            avec = oa[sl]
            bvec = ob[sl]
            cvec = oc[sl]
            svec = on[sl]
            for l in range(L):
                a = avec[l]
                b = bvec[l]
                cc = cvec[l]
                sj = svec[l]
                d = [None] * U
                for u in range(U):
                    d[u] = (sns[u] + sj) + xs[u] * a + ys[u] * b + zs[u] * cc
                m = d[0]
                for u in range(1, U):
                    m = jnp.minimum(m, d[u])
                csl = pl.ds(jc * (L * L) + l * L, L)
                cml[csl] = jnp.minimum(cml[csl], m)
                rms = tuple(jnp.minimum(rms[u], d[u]) for u in range(U))
            return rms

        rms = plsc.parallel_loop(0, P2 // L, carry=(big,) * U, unroll=4)(jc_body)
        for u in range(U):
            sl = pl.ds(base + u * L, L)
            scm = s_buf[3, sl]
            row_acc = row_acc + scm * jnp.maximum(rms[u], 0.0)
            den_acc = den_acc + scm
        return row_acc, den_acc

    zero = jnp.zeros((L,), jnp.float32)
    row_acc, rowden_acc = lax.fori_loop(0, NG, group_body, (zero, zero))

    lanes = lax.iota(jnp.int32, L)

    def red_body(jg):
        acc = zero
        for l in range(L):
            row = cml[pl.ds((jg * L + l) * L, L)]
            acc = jnp.where(lanes == l, jnp.min(row), acc)
        cpart[pl.ds(jg * L, L)] = acc

    plsc.parallel_loop(0, P2 // L, unroll=2)(red_body)

    pltpu.sync_copy(cpart, shared.at[s])
    plsc.subcore_barrier()
    jbase = sl_id * JW
    grp = (s // W) * W

    for p in range(W):
        peer = grp + p

        @pl.when(peer != s)
        def _():
            pltpu.sync_copy(shared.at[peer, pl.ds(jbase, JW)], pbuf)

            def min_body(k, _):
                sl = pl.ds(jbase + k * L, L)
                cpart[sl] = jnp.minimum(cpart[sl], pbuf[pl.ds(k * L, L)])
                return 0

            lax.fori_loop(0, JW // L, min_body, 0)

    def comb_body(k, accs):
        obj_acc, oden_acc = accs
        sl = pl.ds(jbase + k * L, L)
        v = jnp.maximum(cpart[sl], 0.0)
        ocm_vec = ocmv[sl]
        return obj_acc + ocm_vec * v, oden_acc + ocm_vec

    obj_acc, objden_acc = lax.fori_loop(0, JW // L, comb_body, (zero, zero))

    outv[0, :] = row_acc
    outv[1, :] = rowden_acc
    outv[2, :] = obj_acc
    outv[3, :] = objden_acc
    pltpu.sync_copy(outv, out_hbm.at[w])


TI = 768
NI = P1PAD // TI


def _tc_body(x_ref, y_ref, scm_ref, ocm_ref, out_ref, colmin_ref):
    ni = pl.program_id(1)

    x = x_ref[0]
    y = y_ref[0]
    x2 = jnp.sum(x * x, axis=0)[:, None]
    y2 = jnp.sum(y * y, axis=0)[None, :]
    xy = jax.lax.dot_general(
        x, y, (((0,), (0,)), ((), ())), preferred_element_type=jnp.float32
    )
    d = x2 + y2 - 2.0 * xy

    @pl.when(ni == 0)
    def _init():
        colmin_ref[...] = jnp.full_like(colmin_ref, jnp.inf)
        out_ref[0, 0, 0] = 0.0
        out_ref[0, 0, 1] = 0.0
        out_ref[0, 0, 2] = 0.0
        out_ref[0, 0, 3] = 0.0

    scm = scm_ref[0, 0]
    rowmin = jnp.maximum(jnp.min(d, axis=1), 0.0)
    out_ref[0, 0, 0] += jnp.sum(scm * rowmin)
    out_ref[0, 0, 1] += jnp.sum(scm)

    colmin_ref[...] = jnp.minimum(colmin_ref[...], jnp.min(d, axis=0, keepdims=True))

    @pl.when(ni == NI - 1)
    def _fini():
        ocm = ocm_ref[0, 0]
        colmin = jnp.maximum(colmin_ref[0], 0.0)
        out_ref[0, 0, 2] = jnp.sum(ocm * colmin)
        out_ref[0, 0, 3] = jnp.sum(ocm)


def _tc_loss(xpad, object_v, scm, ocm, nb):
    parts = pl.pallas_call(
        _tc_body,
        grid=(nb, NI),
        in_specs=[
            pl.BlockSpec((1, 3, TI), lambda b, i: (b, 0, i)),
            pl.BlockSpec((1, 3, P2), lambda b, i: (b, 0, 0)),
            pl.BlockSpec((1, 1, TI), lambda b, i: (b, 0, i)),
            pl.BlockSpec((1, 1, P2), lambda b, i: (b, 0, 0)),
        ],
        out_specs=pl.BlockSpec((1, 1, 4), lambda b, i: (b, 0, 0),
                               memory_space=pltpu.SMEM),
        out_shape=jax.ShapeDtypeStruct((nb, 1, 4), jnp.float32),
        scratch_shapes=[pltpu.VMEM((1, P2), jnp.float32)],
    )(xpad, object_v, scm, ocm)
    parts = parts[:, 0]
    return jnp.sum(parts[:, 0] / (parts[:, 1] + 1e-6)
                   + parts[:, 2] / (parts[:, 3] + 1e-6))


@jax.jit
def kernel(smpl_v, object_v, smpl_contact_maps, object_contact_maps):
    xt = smpl_v.transpose(0, 2, 1)
    xT = jnp.pad(xt, ((0, 0), (0, 0), (0, P1PAD - P1)),
                 constant_values=PAD_COORD)
    yT = object_v.transpose(0, 2, 1)
    scm = jnp.pad(smpl_contact_maps[..., 0], ((0, 0), (0, P1PAD - P1)))
    ocm = object_contact_maps[..., 0]

    smpl4 = jnp.concatenate([xT[NTC:], scm[NTC:, None, :]], axis=1)
    smpl4 = smpl4.reshape(NSC, 4, W, ROWS).transpose(0, 2, 1, 3)
    smpl4 = smpl4.reshape(2, 16, 4, ROWS).reshape(32, 4, ROWS)

    obj4 = jnp.concatenate([yT[NTC:], ocm[NTC:, None, :]], axis=1)

    parts = _sc_loss(smpl4, obj4)
    parts = parts.sum(axis=2).reshape(NSC, W, 4).sum(axis=1)
    loss_sc = jnp.sum(parts[:, 0] / (parts[:, 1] + 1e-6)
                      + parts[:, 2] / (parts[:, 3] + 1e-6))

    loss_tc = _tc_loss(xT[:NTC], yT[:NTC],
                       scm[:NTC, None, :], ocm[:NTC, None, :], NTC)

    return (loss_tc + loss_sc) / B
